# Initial kernel scaffold; baseline (speedup 1.0000x reference)
import jax
import jax.numpy as jnp
from jax.experimental import pallas as pl


def kernel(x, edge_index, W1, b1, W2, b2):
    raise NotImplementedError("write your pallas kernel here")


# speedup vs baseline: 78.4773x; 78.4773x over previous
import functools

import jax
import jax.numpy as jnp
from jax import lax
from jax.experimental import pallas as pl
from jax.experimental.pallas import tpu as pltpu
from jax.experimental.pallas import tpu_sc as plsc

N = 100000
E = 6400000
NTILES = 16
NCORES = 2
NPAD = 100352
SLICE = NPAD // NTILES
R = NPAD // 128
CH = 1024
KB = CH // 128
NCH = E // CH
PER_CORE = NCH // NCORES
BASE = PER_CORE // NTILES
EXTRA = PER_CORE - BASE * NTILES


def _chunk_range(cid, sid):
    start = cid * PER_CORE + sid * BASE + jnp.minimum(sid, EXTRA)
    n = jnp.where(sid < EXTRA, BASE + 1, BASE)
    return start, n


_SC_MESH = plsc.VectorSubcoreMesh(core_axis_name="c", subcore_axis_name="s")
_SC_PARAMS = pltpu.CompilerParams(use_tc_tiling_on_sc=False)


@functools.partial(
    pl.kernel,
    out_type=jax.ShapeDtypeStruct((NCORES, NPAD), jnp.float32),
    mesh=_SC_MESH,
    compiler_params=_SC_PARAMS,
    scratch_types=[
        pltpu.VMEM((KB, 128), jnp.int32),
        pltpu.VMEM((128,), jnp.float32),
        pltpu.VMEM_SHARED((NPAD,), jnp.float32),
    ],
)
def _deg_pass(dst_h, zero_h, out_h, idx_v, ones_v, acc_s):
    cid = lax.axis_index("c")
    sid = lax.axis_index("s")
    for i in range(8):
        ones_v[pl.ds(i * 16, 16)] = jnp.ones((16,), jnp.float32)
    pltpu.sync_copy(zero_h.at[pl.ds(sid * SLICE, SLICE)],
                    acc_s.at[pl.ds(sid * SLICE, SLICE)])
    plsc.subcore_barrier()
    start, n = _chunk_range(cid, sid)

    def body(i, carry):
        row = (start + i) * KB
        pltpu.sync_copy(dst_h.at[pl.ds(row, KB)], idx_v)
        for j in range(KB):
            pltpu.sync_copy(ones_v, acc_s.at[idx_v.at[j]], add=True)
        return carry

    lax.fori_loop(0, n, body, 0)
    plsc.subcore_barrier()
    pltpu.sync_copy(acc_s.at[pl.ds(sid * SLICE, SLICE)],
                    out_h.at[cid, pl.ds(sid * SLICE, SLICE)])


@functools.partial(
    pl.kernel,
    out_type=jax.ShapeDtypeStruct((NCORES, NPAD), jnp.float32),
    mesh=_SC_MESH,
    compiler_params=_SC_PARAMS,
    scratch_types=[
        pltpu.VMEM((KB, 128), jnp.int32),
        pltpu.VMEM((KB, 128), jnp.int32),
        pltpu.VMEM((KB, 128), jnp.float32),
        pltpu.VMEM_SHARED((NPAD,), jnp.float32),
        pltpu.VMEM_SHARED((NPAD,), jnp.float32),
    ],
)
def _agg1_pass(src_h, dst_h, u_h, zero_h, out_h,
               sidx_v, didx_v, vals_v, u_s, acc_s):
    cid = lax.axis_index("c")
    sid = lax.axis_index("s")
    sl = pl.ds(sid * SLICE, SLICE)
    pltpu.sync_copy(u_h.at[sl], u_s.at[sl])
    pltpu.sync_copy(zero_h.at[sl], acc_s.at[sl])
    plsc.subcore_barrier()
    start, n = _chunk_range(cid, sid)

    def body(i, carry):
        row = (start + i) * KB
        pltpu.sync_copy(src_h.at[pl.ds(row, KB)], sidx_v)
        pltpu.sync_copy(dst_h.at[pl.ds(row, KB)], didx_v)
        for j in range(KB):
            pltpu.sync_copy(u_s.at[sidx_v.at[j]], vals_v.at[j])
        for j in range(KB):
            pltpu.sync_copy(vals_v.at[j], acc_s.at[didx_v.at[j]], add=True)
        return carry

    lax.fori_loop(0, n, body, 0)
    plsc.subcore_barrier()
    pltpu.sync_copy(acc_s.at[sl], out_h.at[cid, sl])


@functools.partial(
    pl.kernel,
    out_type=jax.ShapeDtypeStruct((NCORES, 2, NPAD), jnp.float32),
    mesh=_SC_MESH,
    compiler_params=_SC_PARAMS,
    scratch_types=[
        pltpu.VMEM((KB, 128), jnp.int32),
        pltpu.VMEM((KB, 128), jnp.int32),
        pltpu.VMEM((KB, 128), jnp.float32),
        pltpu.VMEM((KB, 128), jnp.float32),
        pltpu.VMEM_SHARED((NPAD,), jnp.float32),
        pltpu.VMEM_SHARED((NPAD,), jnp.float32),
        pltpu.VMEM_SHARED((NPAD,), jnp.float32),
        pltpu.VMEM_SHARED((NPAD,), jnp.float32),
    ],
)
def _agg2_pass(src_h, dst_h, v_h, zero_h, out_h,
               sidx_v, didx_v, vals0_v, vals1_v, v0_s, v1_s, acc0_s, acc1_s):
    cid = lax.axis_index("c")
    sid = lax.axis_index("s")
    sl = pl.ds(sid * SLICE, SLICE)
    pltpu.sync_copy(v_h.at[0, sl], v0_s.at[sl])
    pltpu.sync_copy(v_h.at[1, sl], v1_s.at[sl])
    pltpu.sync_copy(zero_h.at[sl], acc0_s.at[sl])
    pltpu.sync_copy(zero_h.at[sl], acc1_s.at[sl])
    plsc.subcore_barrier()
    start, n = _chunk_range(cid, sid)

    def body(i, carry):
        row = (start + i) * KB
        pltpu.sync_copy(src_h.at[pl.ds(row, KB)], sidx_v)
        pltpu.sync_copy(dst_h.at[pl.ds(row, KB)], didx_v)
        for j in range(KB):
            pltpu.sync_copy(v0_s.at[sidx_v.at[j]], vals0_v.at[j])
            pltpu.sync_copy(v1_s.at[sidx_v.at[j]], vals1_v.at[j])
        for j in range(KB):
            pltpu.sync_copy(vals0_v.at[j], acc0_s.at[didx_v.at[j]], add=True)
            pltpu.sync_copy(vals1_v.at[j], acc1_s.at[didx_v.at[j]], add=True)
        return carry

    lax.fori_loop(0, n, body, 0)
    plsc.subcore_barrier()
    pltpu.sync_copy(acc0_s.at[sl], out_h.at[cid, 0, sl])
    pltpu.sync_copy(acc1_s.at[sl], out_h.at[cid, 1, sl])


def _dense1_body(degp_ref, x_ref, d_ref, u_ref):
    dp = degp_ref[...]
    deg = dp[0] + dp[1] + 1.0
    d = lax.rsqrt(deg)
    d_ref[...] = d
    u_ref[...] = d * x_ref[...]


def _dense1(degp3, xp3):
    return pl.pallas_call(
        _dense1_body,
        grid=(R // 8,),
        in_specs=[
            pl.BlockSpec((NCORES, 8, 128), lambda i: (0, i, 0)),
            pl.BlockSpec((8, 128), lambda i: (i, 0)),
        ],
        out_specs=[
            pl.BlockSpec((8, 128), lambda i: (i, 0)),
            pl.BlockSpec((8, 128), lambda i: (i, 0)),
        ],
        out_shape=[
            jax.ShapeDtypeStruct((R, 128), jnp.float32),
            jax.ShapeDtypeStruct((R, 128), jnp.float32),
        ],
    )(degp3, xp3)


def _dense2_body(tp_ref, u_ref, d_ref, w1_ref, b1_ref, w2_ref,
                 v0_ref, v1_ref):
    tp = tp_ref[...]
    d = d_ref[...]
    s = d * (tp[0] + tp[1] + u_ref[...])
    z0 = jnp.zeros_like(s)
    z1 = jnp.zeros_like(s)
    for k in range(16):
        h = jnp.maximum(s * w1_ref[0, k] + b1_ref[k], 0.0)
        z0 = z0 + h * w2_ref[k, 0]
        z1 = z1 + h * w2_ref[k, 1]
    v0_ref[...] = d * z0
    v1_ref[...] = d * z1


def _dense2(tp3, u3, d3, w1, b1, w2):
    return pl.pallas_call(
        _dense2_body,
        grid=(R // 8,),
        in_specs=[
            pl.BlockSpec((NCORES, 8, 128), lambda i: (0, i, 0)),
            pl.BlockSpec((8, 128), lambda i: (i, 0)),
            pl.BlockSpec((8, 128), lambda i: (i, 0)),
            pl.BlockSpec(memory_space=pltpu.SMEM),
            pl.BlockSpec(memory_space=pltpu.SMEM),
            pl.BlockSpec(memory_space=pltpu.SMEM),
        ],
        out_specs=[
            pl.BlockSpec((8, 128), lambda i: (i, 0)),
            pl.BlockSpec((8, 128), lambda i: (i, 0)),
        ],
        out_shape=[
            jax.ShapeDtypeStruct((R, 128), jnp.float32),
            jax.ShapeDtypeStruct((R, 128), jnp.float32),
        ],
    )(tp3, u3, d3, w1, b1, w2)


def _dense3_body(t2p_ref, v0_ref, v1_ref, d_ref, b2_ref, o0_ref, o1_ref):
    t2p = t2p_ref[...]
    d = d_ref[...]
    o0_ref[...] = d * (t2p[0, 0] + t2p[1, 0] + v0_ref[...]) + b2_ref[0]
    o1_ref[...] = d * (t2p[0, 1] + t2p[1, 1] + v1_ref[...]) + b2_ref[1]


def _dense3(t2p4, v0, v1, d3, b2):
    return pl.pallas_call(
        _dense3_body,
        grid=(R // 8,),
        in_specs=[
            pl.BlockSpec((NCORES, 2, 8, 128), lambda i: (0, 0, i, 0)),
            pl.BlockSpec((8, 128), lambda i: (i, 0)),
            pl.BlockSpec((8, 128), lambda i: (i, 0)),
            pl.BlockSpec((8, 128), lambda i: (i, 0)),
            pl.BlockSpec(memory_space=pltpu.SMEM),
        ],
        out_specs=[
            pl.BlockSpec((8, 128), lambda i: (i, 0)),
            pl.BlockSpec((8, 128), lambda i: (i, 0)),
        ],
        out_shape=[
            jax.ShapeDtypeStruct((R, 128), jnp.float32),
            jax.ShapeDtypeStruct((R, 128), jnp.float32),
        ],
    )(t2p4, v0, v1, d3, b2)


def kernel(x, edge_index, W1, b1, W2, b2):
    ei = edge_index.astype(jnp.int32)
    src2 = ei[0].reshape(E // 128, 128)
    dst2 = ei[1].reshape(E // 128, 128)
    xp = jnp.pad(x[:, 0], (0, NPAD - N))
    zero1 = jnp.zeros((NPAD,), jnp.float32)

    degp = _deg_pass(dst2, zero1)
    d3, u3 = _dense1(degp.reshape(NCORES, R, 128), xp.reshape(R, 128))

    tp = _agg1_pass(src2, dst2, u3.reshape(NPAD), zero1)
    v0, v1 = _dense2(tp.reshape(NCORES, R, 128), u3, d3, W1, b1, W2)

    v2 = jnp.stack([v0.reshape(NPAD), v1.reshape(NPAD)])
    t2p = _agg2_pass(src2, dst2, v2, zero1)

    o0, o1 = _dense3(t2p.reshape(NCORES, 2, R, 128), v0, v1, d3, b2)
    out = jnp.stack([o0.reshape(NPAD), o1.reshape(NPAD)], axis=-1)
    return out[:N]


# speedup vs baseline: 156.4571x; 1.9937x over previous
import functools

import jax
import jax.numpy as jnp
from jax import lax
from jax.experimental import pallas as pl
from jax.experimental.pallas import tpu as pltpu
from jax.experimental.pallas import tpu_sc as plsc

N = 100000
E = 6400000
NTILES = 16
NCORES = 2
NPAD = 100352
SLICE = NPAD // NTILES
R = NPAD // 128
CH = 2048
NCH = E // CH
CORE0 = (NCH + 1) // 2


def _chunk_range(cid, sid):
    per = jnp.where(cid == 0, CORE0, NCH - CORE0)
    base = cid * CORE0
    b = per // NTILES
    ex = per - b * NTILES
    start = base + sid * b + jnp.minimum(sid, ex)
    n = jnp.where(sid < ex, b + 1, b)
    return start, n


_SC_MESH = plsc.VectorSubcoreMesh(core_axis_name="c", subcore_axis_name="s")
_SC_PARAMS = pltpu.CompilerParams(use_tc_tiling_on_sc=False)


@functools.partial(
    pl.kernel,
    out_type=jax.ShapeDtypeStruct((NCORES, NPAD), jnp.float32),
    mesh=_SC_MESH,
    compiler_params=_SC_PARAMS,
    scratch_types=[
        pltpu.VMEM((CH,), jnp.int32),
        pltpu.VMEM((CH,), jnp.float32),
        pltpu.VMEM_SHARED((NPAD,), jnp.float32),
    ],
)
def _deg_pass(dst_h, zero_h, out_h, idx_v, ones_v, acc_s):
    cid = lax.axis_index("c")
    sid = lax.axis_index("s")
    for i in range(CH // 16):
        ones_v[pl.ds(i * 16, 16)] = jnp.ones((16,), jnp.float32)
    sl = pl.ds(sid * SLICE, SLICE)
    pltpu.sync_copy(zero_h.at[sl], acc_s.at[sl])
    plsc.subcore_barrier()
    start, n = _chunk_range(cid, sid)

    def body(i, carry):
        off = (start + i) * CH
        pltpu.sync_copy(dst_h.at[pl.ds(off, CH)], idx_v)
        pltpu.sync_copy(ones_v, acc_s.at[idx_v], add=True)
        return carry

    lax.fori_loop(0, n, body, 0)
    plsc.subcore_barrier()
    pltpu.sync_copy(acc_s.at[sl], out_h.at[cid, sl])


@functools.partial(
    pl.kernel,
    out_type=jax.ShapeDtypeStruct((NCORES, NPAD), jnp.float32),
    mesh=_SC_MESH,
    compiler_params=_SC_PARAMS,
    scratch_types=[
        pltpu.VMEM((CH,), jnp.int32),
        pltpu.VMEM((CH,), jnp.int32),
        pltpu.VMEM((CH,), jnp.float32),
        pltpu.VMEM_SHARED((NPAD,), jnp.float32),
        pltpu.VMEM_SHARED((NPAD,), jnp.float32),
    ],
)
def _agg1_pass(src_h, dst_h, u_h, zero_h, out_h,
               sidx_v, didx_v, vals_v, u_s, acc_s):
    cid = lax.axis_index("c")
    sid = lax.axis_index("s")
    sl = pl.ds(sid * SLICE, SLICE)
    pltpu.sync_copy(u_h.at[sl], u_s.at[sl])
    pltpu.sync_copy(zero_h.at[sl], acc_s.at[sl])
    plsc.subcore_barrier()
    start, n = _chunk_range(cid, sid)

    def body(i, carry):
        off = (start + i) * CH
        pltpu.sync_copy(src_h.at[pl.ds(off, CH)], sidx_v)
        pltpu.sync_copy(dst_h.at[pl.ds(off, CH)], didx_v)
        pltpu.sync_copy(u_s.at[sidx_v], vals_v)
        pltpu.sync_copy(vals_v, acc_s.at[didx_v], add=True)
        return carry

    lax.fori_loop(0, n, body, 0)
    plsc.subcore_barrier()
    pltpu.sync_copy(acc_s.at[sl], out_h.at[cid, sl])


@functools.partial(
    pl.kernel,
    out_type=jax.ShapeDtypeStruct((NCORES, 2, NPAD), jnp.float32),
    mesh=_SC_MESH,
    compiler_params=_SC_PARAMS,
    scratch_types=[
        pltpu.VMEM((CH,), jnp.int32),
        pltpu.VMEM((CH,), jnp.int32),
        pltpu.VMEM((CH,), jnp.float32),
        pltpu.VMEM((CH,), jnp.float32),
        pltpu.VMEM_SHARED((NPAD,), jnp.float32),
        pltpu.VMEM_SHARED((NPAD,), jnp.float32),
        pltpu.VMEM_SHARED((NPAD,), jnp.float32),
        pltpu.VMEM_SHARED((NPAD,), jnp.float32),
    ],
)
def _agg2_pass(src_h, dst_h, v_h, zero_h, out_h,
               sidx_v, didx_v, vals0_v, vals1_v, v0_s, v1_s, acc0_s, acc1_s):
    cid = lax.axis_index("c")
    sid = lax.axis_index("s")
    sl = pl.ds(sid * SLICE, SLICE)
    pltpu.sync_copy(v_h.at[0, sl], v0_s.at[sl])
    pltpu.sync_copy(v_h.at[1, sl], v1_s.at[sl])
    pltpu.sync_copy(zero_h.at[sl], acc0_s.at[sl])
    pltpu.sync_copy(zero_h.at[sl], acc1_s.at[sl])
    plsc.subcore_barrier()
    start, n = _chunk_range(cid, sid)

    def body(i, carry):
        off = (start + i) * CH
        pltpu.sync_copy(src_h.at[pl.ds(off, CH)], sidx_v)
        pltpu.sync_copy(dst_h.at[pl.ds(off, CH)], didx_v)
        pltpu.sync_copy(v0_s.at[sidx_v], vals0_v)
        pltpu.sync_copy(v1_s.at[sidx_v], vals1_v)
        pltpu.sync_copy(vals0_v, acc0_s.at[didx_v], add=True)
        pltpu.sync_copy(vals1_v, acc1_s.at[didx_v], add=True)
        return carry

    lax.fori_loop(0, n, body, 0)
    plsc.subcore_barrier()
    pltpu.sync_copy(acc0_s.at[sl], out_h.at[cid, 0, sl])
    pltpu.sync_copy(acc1_s.at[sl], out_h.at[cid, 1, sl])


def _dense1_body(degp_ref, x_ref, d_ref, u_ref):
    dp = degp_ref[...]
    deg = dp[0] + dp[1] + 1.0
    d = lax.rsqrt(deg)
    d_ref[...] = d
    u_ref[...] = d * x_ref[...]


def _dense1(degp3, xp3):
    return pl.pallas_call(
        _dense1_body,
        grid=(R // 8,),
        in_specs=[
            pl.BlockSpec((NCORES, 8, 128), lambda i: (0, i, 0)),
            pl.BlockSpec((8, 128), lambda i: (i, 0)),
        ],
        out_specs=[
            pl.BlockSpec((8, 128), lambda i: (i, 0)),
            pl.BlockSpec((8, 128), lambda i: (i, 0)),
        ],
        out_shape=[
            jax.ShapeDtypeStruct((R, 128), jnp.float32),
            jax.ShapeDtypeStruct((R, 128), jnp.float32),
        ],
    )(degp3, xp3)


def _dense2_body(tp_ref, u_ref, d_ref, w1_ref, b1_ref, w2_ref,
                 v0_ref, v1_ref):
    tp = tp_ref[...]
    d = d_ref[...]
    s = d * (tp[0] + tp[1] + u_ref[...])
    z0 = jnp.zeros_like(s)
    z1 = jnp.zeros_like(s)
    for k in range(16):
        h = jnp.maximum(s * w1_ref[0, k] + b1_ref[k], 0.0)
        z0 = z0 + h * w2_ref[k, 0]
        z1 = z1 + h * w2_ref[k, 1]
    v0_ref[...] = d * z0
    v1_ref[...] = d * z1


def _dense2(tp3, u3, d3, w1, b1, w2):
    return pl.pallas_call(
        _dense2_body,
        grid=(R // 8,),
        in_specs=[
            pl.BlockSpec((NCORES, 8, 128), lambda i: (0, i, 0)),
            pl.BlockSpec((8, 128), lambda i: (i, 0)),
            pl.BlockSpec((8, 128), lambda i: (i, 0)),
            pl.BlockSpec(memory_space=pltpu.SMEM),
            pl.BlockSpec(memory_space=pltpu.SMEM),
            pl.BlockSpec(memory_space=pltpu.SMEM),
        ],
        out_specs=[
            pl.BlockSpec((8, 128), lambda i: (i, 0)),
            pl.BlockSpec((8, 128), lambda i: (i, 0)),
        ],
        out_shape=[
            jax.ShapeDtypeStruct((R, 128), jnp.float32),
            jax.ShapeDtypeStruct((R, 128), jnp.float32),
        ],
    )(tp3, u3, d3, w1, b1, w2)


def _dense3_body(t2p_ref, v0_ref, v1_ref, d_ref, b2_ref, o0_ref, o1_ref):
    t2p = t2p_ref[...]
    d = d_ref[...]
    o0_ref[...] = d * (t2p[0, 0] + t2p[1, 0] + v0_ref[...]) + b2_ref[0]
    o1_ref[...] = d * (t2p[0, 1] + t2p[1, 1] + v1_ref[...]) + b2_ref[1]


def _dense3(t2p4, v0, v1, d3, b2):
    return pl.pallas_call(
        _dense3_body,
        grid=(R // 8,),
        in_specs=[
            pl.BlockSpec((NCORES, 2, 8, 128), lambda i: (0, 0, i, 0)),
            pl.BlockSpec((8, 128), lambda i: (i, 0)),
            pl.BlockSpec((8, 128), lambda i: (i, 0)),
            pl.BlockSpec((8, 128), lambda i: (i, 0)),
            pl.BlockSpec(memory_space=pltpu.SMEM),
        ],
        out_specs=[
            pl.BlockSpec((8, 128), lambda i: (i, 0)),
            pl.BlockSpec((8, 128), lambda i: (i, 0)),
        ],
        out_shape=[
            jax.ShapeDtypeStruct((R, 128), jnp.float32),
            jax.ShapeDtypeStruct((R, 128), jnp.float32),
        ],
    )(t2p4, v0, v1, d3, b2)


def kernel(x, edge_index, W1, b1, W2, b2):
    ei = edge_index.astype(jnp.int32)
    src1 = ei[0]
    dst1 = ei[1]
    xp = jnp.pad(x[:, 0], (0, NPAD - N))
    zero1 = jnp.zeros((NPAD,), jnp.float32)

    degp = _deg_pass(dst1, zero1)
    d3, u3 = _dense1(degp.reshape(NCORES, R, 128), xp.reshape(R, 128))

    tp = _agg1_pass(src1, dst1, u3.reshape(NPAD), zero1)
    v0, v1 = _dense2(tp.reshape(NCORES, R, 128), u3, d3, W1, b1, W2)

    v2 = jnp.stack([v0.reshape(NPAD), v1.reshape(NPAD)])
    t2p = _agg2_pass(src1, dst1, v2, zero1)

    o0, o1 = _dense3(t2p.reshape(NCORES, 2, R, 128), v0, v1, d3, b2)
    out = jnp.stack([o0.reshape(NPAD), o1.reshape(NPAD)], axis=-1)
    return out[:N]


# speedup vs baseline: 237.2919x; 1.5167x over previous
import functools

import jax
import jax.numpy as jnp
from jax import lax
from jax.experimental import pallas as pl
from jax.experimental.pallas import tpu as pltpu
from jax.experimental.pallas import tpu_sc as plsc

N = 100000
E = 6400000
NTILES = 16
NCORES = 2
NPAD = 100352
SLICE = NPAD // NTILES
R = NPAD // 128
CH = 2048
NCH = E // CH
CORE0 = (NCH + 1) // 2


def _chunk_range(cid, sid):
    per = jnp.where(cid == 0, CORE0, NCH - CORE0)
    base = cid * CORE0
    b = per // NTILES
    ex = per - b * NTILES
    start = base + sid * b + jnp.minimum(sid, ex)
    n = jnp.where(sid < ex, b + 1, b)
    return start, n


_SC_MESH = plsc.VectorSubcoreMesh(core_axis_name="c", subcore_axis_name="s")
_SC_PARAMS = pltpu.CompilerParams(use_tc_tiling_on_sc=False)


@functools.partial(
    pl.kernel,
    out_type=jax.ShapeDtypeStruct((NCORES, NPAD), jnp.float32),
    mesh=_SC_MESH,
    compiler_params=_SC_PARAMS,
    scratch_types=[
        pltpu.VMEM((CH,), jnp.int32),
        pltpu.VMEM((CH,), jnp.int32),
        pltpu.VMEM((CH,), jnp.int32),
        pltpu.VMEM((CH,), jnp.float32),
        pltpu.VMEM_SHARED((NPAD,), jnp.float32),
        pltpu.SemaphoreType.DMA,
        pltpu.SemaphoreType.DMA,
        pltpu.SemaphoreType.DMA,
        pltpu.SemaphoreType.DMA,
        pltpu.SemaphoreType.DMA,
        pltpu.SemaphoreType.DMA,
    ],
)
def _deg_pass(dst_h, zero_h, out_h, di0, di1, di2, ones_v, acc_s,
              semi0, semi1, semi2, sems0, sems1, sems2):
    didx = [di0, di1, di2]
    semi = [semi0, semi1, semi2]
    sems = [sems0, sems1, sems2]
    cid = lax.axis_index("c")
    sid = lax.axis_index("s")
    for i in range(CH // 16):
        ones_v[pl.ds(i * 16, 16)] = jnp.ones((16,), jnp.float32)
    sl = pl.ds(sid * SLICE, SLICE)
    pltpu.sync_copy(zero_h.at[sl], acc_s.at[sl])
    plsc.subcore_barrier()
    start, n = _chunk_range(cid, sid)

    def issue_idx(i, k):
        pltpu.async_copy(dst_h.at[pl.ds((start + i) * CH, CH)], didx[k], semi[k])

    issue_idx(0, 0)

    def body(g, carry):
        for k in range(3):
            i = g * 3 + k
            k2 = (k + 1) % 3

            @pl.when(i < n)
            def _(i=i, k=k, k2=k2):
                @pl.when(i >= 2)
                def _():
                    pltpu.make_async_copy(ones_v, acc_s.at[didx[k2]], sems[k2]).wait()

                @pl.when(i + 1 < n)
                def _():
                    issue_idx(i + 1, k2)

                pltpu.make_async_copy(dst_h.at[pl.ds(0, CH)], didx[k], semi[k]).wait()
                pltpu.async_copy(ones_v, acc_s.at[didx[k]], sems[k], add=True)

        return carry

    lax.fori_loop(0, (n + 2) // 3, body, 0)
    nm = n % 3
    for k in range(3):
        @pl.when(nm != k)
        def _(k=k):
            pltpu.make_async_copy(ones_v, acc_s.at[didx[k]], sems[k]).wait()
    plsc.subcore_barrier()
    pltpu.sync_copy(acc_s.at[sl], out_h.at[cid, sl])


@functools.partial(
    pl.kernel,
    out_type=jax.ShapeDtypeStruct((NCORES, NPAD), jnp.float32),
    mesh=_SC_MESH,
    compiler_params=_SC_PARAMS,
    scratch_types=[
        pltpu.VMEM((CH,), jnp.int32),
        pltpu.VMEM((CH,), jnp.int32),
        pltpu.VMEM((CH,), jnp.int32),
        pltpu.VMEM((CH,), jnp.int32),
        pltpu.VMEM((CH,), jnp.int32),
        pltpu.VMEM((CH,), jnp.int32),
        pltpu.VMEM((CH,), jnp.float32),
        pltpu.VMEM((CH,), jnp.float32),
        pltpu.VMEM((CH,), jnp.float32),
        pltpu.VMEM_SHARED((NPAD,), jnp.float32),
        pltpu.VMEM_SHARED((NPAD,), jnp.float32),
        pltpu.SemaphoreType.DMA,
        pltpu.SemaphoreType.DMA,
        pltpu.SemaphoreType.DMA,
        pltpu.SemaphoreType.DMA,
        pltpu.SemaphoreType.DMA,
        pltpu.SemaphoreType.DMA,
    ],
)
def _agg1_pass(src_h, dst_h, u_h, zero_h, out_h,
               si0, si1, si2, di0, di1, di2, va0, va1, va2, u_s, acc_s,
               semi0, semi1, semi2, sems0, sems1, sems2):
    sidx = [si0, si1, si2]
    didx = [di0, di1, di2]
    vals = [va0, va1, va2]
    semi = [semi0, semi1, semi2]
    sems = [sems0, sems1, sems2]
    cid = lax.axis_index("c")
    sid = lax.axis_index("s")
    sl = pl.ds(sid * SLICE, SLICE)
    pltpu.sync_copy(u_h.at[sl], u_s.at[sl])
    pltpu.sync_copy(zero_h.at[sl], acc_s.at[sl])
    plsc.subcore_barrier()
    start, n = _chunk_range(cid, sid)

    def issue_idx(i, k):
        off = (start + i) * CH
        pltpu.async_copy(src_h.at[pl.ds(off, CH)], sidx[k], semi[k])
        pltpu.async_copy(dst_h.at[pl.ds(off, CH)], didx[k], semi[k])

    issue_idx(0, 0)

    def body(g, carry):
        for k in range(3):
            i = g * 3 + k
            k2 = (k + 1) % 3

            @pl.when(i < n)
            def _(i=i, k=k, k2=k2):
                @pl.when(i >= 2)
                def _():
                    pltpu.make_async_copy(vals[k2], acc_s.at[didx[k2]], sems[k2]).wait()

                @pl.when(i + 1 < n)
                def _():
                    issue_idx(i + 1, k2)

                pltpu.make_async_copy(src_h.at[pl.ds(0, CH)], sidx[k], semi[k]).wait()
                pltpu.make_async_copy(dst_h.at[pl.ds(0, CH)], didx[k], semi[k]).wait()
                pltpu.sync_copy(u_s.at[sidx[k]], vals[k])
                pltpu.async_copy(vals[k], acc_s.at[didx[k]], sems[k], add=True)

        return carry

    lax.fori_loop(0, (n + 2) // 3, body, 0)
    nm = n % 3
    for k in range(3):
        @pl.when(nm != k)
        def _(k=k):
            pltpu.make_async_copy(vals[k], acc_s.at[didx[k]], sems[k]).wait()
    plsc.subcore_barrier()
    pltpu.sync_copy(acc_s.at[sl], out_h.at[cid, sl])


@functools.partial(
    pl.kernel,
    out_type=jax.ShapeDtypeStruct((NCORES, 2, NPAD), jnp.float32),
    mesh=_SC_MESH,
    compiler_params=_SC_PARAMS,
    scratch_types=[
        pltpu.VMEM((CH,), jnp.int32),
        pltpu.VMEM((CH,), jnp.int32),
        pltpu.VMEM((CH,), jnp.int32),
        pltpu.VMEM((CH,), jnp.int32),
        pltpu.VMEM((CH,), jnp.int32),
        pltpu.VMEM((CH,), jnp.int32),
        pltpu.VMEM((CH,), jnp.float32),
        pltpu.VMEM((CH,), jnp.float32),
        pltpu.VMEM((CH,), jnp.float32),
        pltpu.VMEM((CH,), jnp.float32),
        pltpu.VMEM((CH,), jnp.float32),
        pltpu.VMEM((CH,), jnp.float32),
        pltpu.VMEM_SHARED((NPAD,), jnp.float32),
        pltpu.VMEM_SHARED((NPAD,), jnp.float32),
        pltpu.VMEM_SHARED((NPAD,), jnp.float32),
        pltpu.VMEM_SHARED((NPAD,), jnp.float32),
        pltpu.SemaphoreType.DMA,
        pltpu.SemaphoreType.DMA,
        pltpu.SemaphoreType.DMA,
        pltpu.SemaphoreType.DMA,
        pltpu.SemaphoreType.DMA,
        pltpu.SemaphoreType.DMA,
    ],
)
def _agg2_pass(src_h, dst_h, v_h, zero_h, out_h,
               si0, si1, si2, di0, di1, di2,
               va00, va01, va02, va10, va11, va12,
               v0_s, v1_s, acc0_s, acc1_s,
               semi0, semi1, semi2, sems0, sems1, sems2):
    sidx = [si0, si1, si2]
    didx = [di0, di1, di2]
    vals0 = [va00, va01, va02]
    vals1 = [va10, va11, va12]
    semi = [semi0, semi1, semi2]
    sems = [sems0, sems1, sems2]
    cid = lax.axis_index("c")
    sid = lax.axis_index("s")
    sl = pl.ds(sid * SLICE, SLICE)
    pltpu.sync_copy(v_h.at[0, sl], v0_s.at[sl])
    pltpu.sync_copy(v_h.at[1, sl], v1_s.at[sl])
    pltpu.sync_copy(zero_h.at[sl], acc0_s.at[sl])
    pltpu.sync_copy(zero_h.at[sl], acc1_s.at[sl])
    plsc.subcore_barrier()
    start, n = _chunk_range(cid, sid)

    def issue_idx(i, k):
        off = (start + i) * CH
        pltpu.async_copy(src_h.at[pl.ds(off, CH)], sidx[k], semi[k])
        pltpu.async_copy(dst_h.at[pl.ds(off, CH)], didx[k], semi[k])

    def wait_scatter(k):
        pltpu.make_async_copy(vals0[k], acc0_s.at[didx[k]], sems[k]).wait()
        pltpu.make_async_copy(vals1[k], acc1_s.at[didx[k]], sems[k]).wait()

    issue_idx(0, 0)

    def body(g, carry):
        for k in range(3):
            i = g * 3 + k
            k2 = (k + 1) % 3

            @pl.when(i < n)
            def _(i=i, k=k, k2=k2):
                @pl.when(i >= 2)
                def _():
                    wait_scatter(k2)

                @pl.when(i + 1 < n)
                def _():
                    issue_idx(i + 1, k2)

                pltpu.make_async_copy(src_h.at[pl.ds(0, CH)], sidx[k], semi[k]).wait()
                pltpu.make_async_copy(dst_h.at[pl.ds(0, CH)], didx[k], semi[k]).wait()
                pltpu.sync_copy(v0_s.at[sidx[k]], vals0[k])
                pltpu.sync_copy(v1_s.at[sidx[k]], vals1[k])
                pltpu.async_copy(vals0[k], acc0_s.at[didx[k]], sems[k], add=True)
                pltpu.async_copy(vals1[k], acc1_s.at[didx[k]], sems[k], add=True)

        return carry

    lax.fori_loop(0, (n + 2) // 3, body, 0)
    nm = n % 3
    for k in range(3):
        @pl.when(nm != k)
        def _(k=k):
            wait_scatter(k)
    plsc.subcore_barrier()
    pltpu.sync_copy(acc0_s.at[sl], out_h.at[cid, 0, sl])
    pltpu.sync_copy(acc1_s.at[sl], out_h.at[cid, 1, sl])


def _dense1_body(degp_ref, x_ref, d_ref, u_ref):
    dp = degp_ref[...]
    deg = dp[0] + dp[1] + 1.0
    d = lax.rsqrt(deg)
    d_ref[...] = d
    u_ref[...] = d * x_ref[...]


def _dense1(degp3, xp3):
    return pl.pallas_call(
        _dense1_body,
        grid=(R // 8,),
        in_specs=[
            pl.BlockSpec((NCORES, 8, 128), lambda i: (0, i, 0)),
            pl.BlockSpec((8, 128), lambda i: (i, 0)),
        ],
        out_specs=[
            pl.BlockSpec((8, 128), lambda i: (i, 0)),
            pl.BlockSpec((8, 128), lambda i: (i, 0)),
        ],
        out_shape=[
            jax.ShapeDtypeStruct((R, 128), jnp.float32),
            jax.ShapeDtypeStruct((R, 128), jnp.float32),
        ],
    )(degp3, xp3)


def _dense2_body(tp_ref, u_ref, d_ref, w1_ref, b1_ref, w2_ref,
                 v0_ref, v1_ref):
    tp = tp_ref[...]
    d = d_ref[...]
    s = d * (tp[0] + tp[1] + u_ref[...])
    z0 = jnp.zeros_like(s)
    z1 = jnp.zeros_like(s)
    for k in range(16):
        h = jnp.maximum(s * w1_ref[0, k] + b1_ref[k], 0.0)
        z0 = z0 + h * w2_ref[k, 0]
        z1 = z1 + h * w2_ref[k, 1]
    v0_ref[...] = d * z0
    v1_ref[...] = d * z1


def _dense2(tp3, u3, d3, w1, b1, w2):
    return pl.pallas_call(
        _dense2_body,
        grid=(R // 8,),
        in_specs=[
            pl.BlockSpec((NCORES, 8, 128), lambda i: (0, i, 0)),
            pl.BlockSpec((8, 128), lambda i: (i, 0)),
            pl.BlockSpec((8, 128), lambda i: (i, 0)),
            pl.BlockSpec(memory_space=pltpu.SMEM),
            pl.BlockSpec(memory_space=pltpu.SMEM),
            pl.BlockSpec(memory_space=pltpu.SMEM),
        ],
        out_specs=[
            pl.BlockSpec((8, 128), lambda i: (i, 0)),
            pl.BlockSpec((8, 128), lambda i: (i, 0)),
        ],
        out_shape=[
            jax.ShapeDtypeStruct((R, 128), jnp.float32),
            jax.ShapeDtypeStruct((R, 128), jnp.float32),
        ],
    )(tp3, u3, d3, w1, b1, w2)


def _dense3_body(t2p_ref, v0_ref, v1_ref, d_ref, b2_ref, o0_ref, o1_ref):
    t2p = t2p_ref[...]
    d = d_ref[...]
    o0_ref[...] = d * (t2p[0, 0] + t2p[1, 0] + v0_ref[...]) + b2_ref[0]
    o1_ref[...] = d * (t2p[0, 1] + t2p[1, 1] + v1_ref[...]) + b2_ref[1]


def _dense3(t2p4, v0, v1, d3, b2):
    return pl.pallas_call(
        _dense3_body,
        grid=(R // 8,),
        in_specs=[
            pl.BlockSpec((NCORES, 2, 8, 128), lambda i: (0, 0, i, 0)),
            pl.BlockSpec((8, 128), lambda i: (i, 0)),
            pl.BlockSpec((8, 128), lambda i: (i, 0)),
            pl.BlockSpec((8, 128), lambda i: (i, 0)),
            pl.BlockSpec(memory_space=pltpu.SMEM),
        ],
        out_specs=[
            pl.BlockSpec((8, 128), lambda i: (i, 0)),
            pl.BlockSpec((8, 128), lambda i: (i, 0)),
        ],
        out_shape=[
            jax.ShapeDtypeStruct((R, 128), jnp.float32),
            jax.ShapeDtypeStruct((R, 128), jnp.float32),
        ],
    )(t2p4, v0, v1, d3, b2)


def kernel(x, edge_index, W1, b1, W2, b2):
    ei = edge_index.astype(jnp.int32)
    src1 = ei[0]
    dst1 = ei[1]
    xp = jnp.pad(x[:, 0], (0, NPAD - N))
    zero1 = jnp.zeros((NPAD,), jnp.float32)

    degp = _deg_pass(dst1, zero1)
    d3, u3 = _dense1(degp.reshape(NCORES, R, 128), xp.reshape(R, 128))

    tp = _agg1_pass(src1, dst1, u3.reshape(NPAD), zero1)
    v0, v1 = _dense2(tp.reshape(NCORES, R, 128), u3, d3, W1, b1, W2)

    v2 = jnp.stack([v0.reshape(NPAD), v1.reshape(NPAD)])
    t2p = _agg2_pass(src1, dst1, v2, zero1)

    o0, o1 = _dense3(t2p.reshape(NCORES, 2, R, 128), v0, v1, d3, b2)
    out = jnp.stack([o0.reshape(NPAD), o1.reshape(NPAD)], axis=-1)
    return out[:N]


# speedup vs baseline: 240.7374x; 1.0145x over previous
import functools

import jax
import jax.numpy as jnp
from jax import lax
from jax.experimental import pallas as pl
from jax.experimental.pallas import tpu as pltpu
from jax.experimental.pallas import tpu_sc as plsc

N = 100000
E = 6400000
NTILES = 16
NCORES = 2
NPAD = 100352
SLICE = NPAD // NTILES
R = NPAD // 128
CH = 2048
NCH = E // CH
CORE0 = (NCH + 1) // 2


def _chunk_range(cid, sid):
    per = jnp.where(cid == 0, CORE0, NCH - CORE0)
    base = cid * CORE0
    b = per // NTILES
    ex = per - b * NTILES
    start = base + sid * b + jnp.minimum(sid, ex)
    n = jnp.where(sid < ex, b + 1, b)
    return start, n


_SC_MESH = plsc.VectorSubcoreMesh(core_axis_name="c", subcore_axis_name="s")
_SC_PARAMS = pltpu.CompilerParams(use_tc_tiling_on_sc=False)


@functools.partial(
    pl.kernel,
    out_type=jax.ShapeDtypeStruct((NCORES, NPAD), jnp.float32),
    mesh=_SC_MESH,
    compiler_params=_SC_PARAMS,
    scratch_types=[
        pltpu.VMEM((CH,), jnp.int32),
        pltpu.VMEM((CH,), jnp.int32),
        pltpu.VMEM((CH,), jnp.int32),
        pltpu.VMEM((CH,), jnp.float32),
        pltpu.VMEM_SHARED((NPAD,), jnp.float32),
        pltpu.SemaphoreType.DMA,
        pltpu.SemaphoreType.DMA,
        pltpu.SemaphoreType.DMA,
        pltpu.SemaphoreType.DMA,
        pltpu.SemaphoreType.DMA,
        pltpu.SemaphoreType.DMA,
    ],
)
def _deg_pass(dst_h, zero_h, out_h, di0, di1, di2, ones_v, acc_s,
              semi0, semi1, semi2, sems0, sems1, sems2):
    didx = [di0, di1, di2]
    semi = [semi0, semi1, semi2]
    sems = [sems0, sems1, sems2]
    cid = lax.axis_index("c")
    sid = lax.axis_index("s")
    for i in range(CH // 16):
        ones_v[pl.ds(i * 16, 16)] = jnp.ones((16,), jnp.float32)
    sl = pl.ds(sid * SLICE, SLICE)
    pltpu.sync_copy(zero_h.at[sl], acc_s.at[sl])
    plsc.subcore_barrier()
    start, n = _chunk_range(cid, sid)

    def issue_idx(i, k):
        pltpu.async_copy(dst_h.at[pl.ds((start + i) * CH, CH)], didx[k], semi[k])

    issue_idx(0, 0)

    def body(g, carry):
        for k in range(3):
            i = g * 3 + k
            k2 = (k + 1) % 3

            @pl.when(i < n)
            def _(i=i, k=k, k2=k2):
                @pl.when(i >= 2)
                def _():
                    pltpu.make_async_copy(ones_v, acc_s.at[didx[k2]], sems[k2]).wait()

                @pl.when(i + 1 < n)
                def _():
                    issue_idx(i + 1, k2)

                pltpu.make_async_copy(dst_h.at[pl.ds(0, CH)], didx[k], semi[k]).wait()
                pltpu.async_copy(ones_v, acc_s.at[didx[k]], sems[k], add=True)

        return carry

    lax.fori_loop(0, (n + 2) // 3, body, 0)
    nm = n % 3
    for k in range(3):
        @pl.when(nm != k)
        def _(k=k):
            pltpu.make_async_copy(ones_v, acc_s.at[didx[k]], sems[k]).wait()
    plsc.subcore_barrier()
    pltpu.sync_copy(acc_s.at[sl], out_h.at[cid, sl])


@functools.partial(
    pl.kernel,
    out_type=jax.ShapeDtypeStruct((NCORES, NPAD), jnp.float32),
    mesh=_SC_MESH,
    compiler_params=_SC_PARAMS,
    scratch_types=[
        pltpu.VMEM((CH,), jnp.int32),
        pltpu.VMEM((CH,), jnp.int32),
        pltpu.VMEM((CH,), jnp.int32),
        pltpu.VMEM((CH,), jnp.int32),
        pltpu.VMEM((CH,), jnp.int32),
        pltpu.VMEM((CH,), jnp.int32),
        pltpu.VMEM((CH,), jnp.float32),
        pltpu.VMEM((CH,), jnp.float32),
        pltpu.VMEM((CH,), jnp.float32),
        pltpu.VMEM_SHARED((NPAD,), jnp.float32),
        pltpu.VMEM_SHARED((NPAD,), jnp.float32),
        pltpu.SemaphoreType.DMA,
        pltpu.SemaphoreType.DMA,
        pltpu.SemaphoreType.DMA,
        pltpu.SemaphoreType.DMA,
        pltpu.SemaphoreType.DMA,
        pltpu.SemaphoreType.DMA,
    ],
)
def _agg1_pass(src_h, dst_h, u_h, zero_h, out_h,
               si0, si1, si2, di0, di1, di2, va0, va1, va2, u_s, acc_s,
               semi0, semi1, semi2, sems0, sems1, sems2):
    sidx = [si0, si1, si2]
    didx = [di0, di1, di2]
    vals = [va0, va1, va2]
    semi = [semi0, semi1, semi2]
    sems = [sems0, sems1, sems2]
    cid = lax.axis_index("c")
    sid = lax.axis_index("s")
    sl = pl.ds(sid * SLICE, SLICE)
    pltpu.sync_copy(u_h.at[sl], u_s.at[sl])
    pltpu.sync_copy(zero_h.at[sl], acc_s.at[sl])
    plsc.subcore_barrier()
    start, n = _chunk_range(cid, sid)

    def issue_idx(i, k):
        off = (start + i) * CH
        pltpu.async_copy(src_h.at[pl.ds(off, CH)], sidx[k], semi[k])
        pltpu.async_copy(dst_h.at[pl.ds(off, CH)], didx[k], semi[k])

    issue_idx(0, 0)

    def body(g, carry):
        for k in range(3):
            i = g * 3 + k
            k2 = (k + 1) % 3

            @pl.when(i < n)
            def _(i=i, k=k, k2=k2):
                @pl.when(i >= 2)
                def _():
                    pltpu.make_async_copy(vals[k2], acc_s.at[didx[k2]], sems[k2]).wait()

                @pl.when(i + 1 < n)
                def _():
                    issue_idx(i + 1, k2)

                pltpu.make_async_copy(src_h.at[pl.ds(0, CH)], sidx[k], semi[k]).wait()
                pltpu.make_async_copy(dst_h.at[pl.ds(0, CH)], didx[k], semi[k]).wait()
                pltpu.sync_copy(u_s.at[sidx[k]], vals[k])
                pltpu.async_copy(vals[k], acc_s.at[didx[k]], sems[k], add=True)

        return carry

    lax.fori_loop(0, (n + 2) // 3, body, 0)
    nm = n % 3
    for k in range(3):
        @pl.when(nm != k)
        def _(k=k):
            pltpu.make_async_copy(vals[k], acc_s.at[didx[k]], sems[k]).wait()
    plsc.subcore_barrier()
    pltpu.sync_copy(acc_s.at[sl], out_h.at[cid, sl])


@functools.partial(
    pl.kernel,
    out_type=jax.ShapeDtypeStruct((NCORES, 2, NPAD), jnp.float32),
    mesh=_SC_MESH,
    compiler_params=_SC_PARAMS,
    scratch_types=[
        pltpu.VMEM((CH,), jnp.int32),
        pltpu.VMEM((CH,), jnp.int32),
        pltpu.VMEM((CH,), jnp.int32),
        pltpu.VMEM((CH,), jnp.int32),
        pltpu.VMEM((CH,), jnp.int32),
        pltpu.VMEM((CH,), jnp.int32),
        pltpu.VMEM((CH,), jnp.float32),
        pltpu.VMEM((CH,), jnp.float32),
        pltpu.VMEM((CH,), jnp.float32),
        pltpu.VMEM((CH,), jnp.float32),
        pltpu.VMEM((CH,), jnp.float32),
        pltpu.VMEM((CH,), jnp.float32),
        pltpu.VMEM_SHARED((NPAD,), jnp.float32),
        pltpu.VMEM_SHARED((NPAD,), jnp.float32),
        pltpu.VMEM_SHARED((NPAD,), jnp.float32),
        pltpu.VMEM_SHARED((NPAD,), jnp.float32),
        pltpu.SemaphoreType.DMA,
        pltpu.SemaphoreType.DMA,
        pltpu.SemaphoreType.DMA,
        pltpu.SemaphoreType.DMA,
        pltpu.SemaphoreType.DMA,
        pltpu.SemaphoreType.DMA,
        pltpu.SemaphoreType.DMA,
    ],
)
def _agg2_pass(src_h, dst_h, v_h, zero_h, out_h,
               si0, si1, si2, di0, di1, di2,
               va00, va01, va02, va10, va11, va12,
               v0_s, v1_s, acc0_s, acc1_s,
               semi0, semi1, semi2, sems0, sems1, sems2, semg):
    sidx = [si0, si1, si2]
    didx = [di0, di1, di2]
    vals0 = [va00, va01, va02]
    vals1 = [va10, va11, va12]
    semi = [semi0, semi1, semi2]
    sems = [sems0, sems1, sems2]
    cid = lax.axis_index("c")
    sid = lax.axis_index("s")
    sl = pl.ds(sid * SLICE, SLICE)
    pltpu.sync_copy(v_h.at[0, sl], v0_s.at[sl])
    pltpu.sync_copy(v_h.at[1, sl], v1_s.at[sl])
    pltpu.sync_copy(zero_h.at[sl], acc0_s.at[sl])
    pltpu.sync_copy(zero_h.at[sl], acc1_s.at[sl])
    plsc.subcore_barrier()
    start, n = _chunk_range(cid, sid)

    def issue_idx(i, k):
        off = (start + i) * CH
        pltpu.async_copy(src_h.at[pl.ds(off, CH)], sidx[k], semi[k])
        pltpu.async_copy(dst_h.at[pl.ds(off, CH)], didx[k], semi[k])

    def wait_scatter(k):
        pltpu.make_async_copy(vals0[k], acc0_s.at[didx[k]], sems[k]).wait()
        pltpu.make_async_copy(vals1[k], acc1_s.at[didx[k]], sems[k]).wait()

    issue_idx(0, 0)

    def body(g, carry):
        for k in range(3):
            i = g * 3 + k
            k2 = (k + 1) % 3

            @pl.when(i < n)
            def _(i=i, k=k, k2=k2):
                @pl.when(i >= 2)
                def _():
                    wait_scatter(k2)

                @pl.when(i + 1 < n)
                def _():
                    issue_idx(i + 1, k2)

                pltpu.make_async_copy(src_h.at[pl.ds(0, CH)], sidx[k], semi[k]).wait()
                pltpu.make_async_copy(dst_h.at[pl.ds(0, CH)], didx[k], semi[k]).wait()
                pltpu.async_copy(v0_s.at[sidx[k]], vals0[k], semg)
                pltpu.async_copy(v1_s.at[sidx[k]], vals1[k], semg)
                pltpu.make_async_copy(v0_s.at[sidx[k]], vals0[k], semg).wait()
                pltpu.make_async_copy(v1_s.at[sidx[k]], vals1[k], semg).wait()
                pltpu.async_copy(vals0[k], acc0_s.at[didx[k]], sems[k], add=True)
                pltpu.async_copy(vals1[k], acc1_s.at[didx[k]], sems[k], add=True)

        return carry

    lax.fori_loop(0, (n + 2) // 3, body, 0)
    nm = n % 3
    for k in range(3):
        @pl.when(nm != k)
        def _(k=k):
            wait_scatter(k)
    plsc.subcore_barrier()
    pltpu.sync_copy(acc0_s.at[sl], out_h.at[cid, 0, sl])
    pltpu.sync_copy(acc1_s.at[sl], out_h.at[cid, 1, sl])


def _dense1_body(degp_ref, x_ref, d_ref, u_ref):
    dp = degp_ref[...]
    deg = dp[0] + dp[1] + 1.0
    d = lax.rsqrt(deg)
    d_ref[...] = d
    u_ref[...] = d * x_ref[...]


def _dense1(degp3, xp3):
    return pl.pallas_call(
        _dense1_body,
        grid=(R // 8,),
        in_specs=[
            pl.BlockSpec((NCORES, 8, 128), lambda i: (0, i, 0)),
            pl.BlockSpec((8, 128), lambda i: (i, 0)),
        ],
        out_specs=[
            pl.BlockSpec((8, 128), lambda i: (i, 0)),
            pl.BlockSpec((8, 128), lambda i: (i, 0)),
        ],
        out_shape=[
            jax.ShapeDtypeStruct((R, 128), jnp.float32),
            jax.ShapeDtypeStruct((R, 128), jnp.float32),
        ],
    )(degp3, xp3)


def _dense2_body(tp_ref, u_ref, d_ref, w1_ref, b1_ref, w2_ref, v_ref):
    tp = tp_ref[...]
    d = d_ref[...]
    s = d * (tp[0] + tp[1] + u_ref[...])
    z0 = jnp.zeros_like(s)
    z1 = jnp.zeros_like(s)
    for k in range(16):
        h = jnp.maximum(s * w1_ref[0, k] + b1_ref[k], 0.0)
        z0 = z0 + h * w2_ref[k, 0]
        z1 = z1 + h * w2_ref[k, 1]
    v_ref[0] = d * z0
    v_ref[1] = d * z1


def _dense2(tp3, u3, d3, w1, b1, w2):
    return pl.pallas_call(
        _dense2_body,
        grid=(R // 8,),
        in_specs=[
            pl.BlockSpec((NCORES, 8, 128), lambda i: (0, i, 0)),
            pl.BlockSpec((8, 128), lambda i: (i, 0)),
            pl.BlockSpec((8, 128), lambda i: (i, 0)),
            pl.BlockSpec(memory_space=pltpu.SMEM),
            pl.BlockSpec(memory_space=pltpu.SMEM),
            pl.BlockSpec(memory_space=pltpu.SMEM),
        ],
        out_specs=pl.BlockSpec((2, 8, 128), lambda i: (0, i, 0)),
        out_shape=jax.ShapeDtypeStruct((2, R, 128), jnp.float32),
    )(tp3, u3, d3, w1, b1, w2)


def _dense3_body(t2p_ref, v_ref, d_ref, b2_ref, o0_ref, o1_ref):
    t2p = t2p_ref[...]
    v = v_ref[...]
    d = d_ref[...]
    o0_ref[...] = d * (t2p[0, 0] + t2p[1, 0] + v[0]) + b2_ref[0]
    o1_ref[...] = d * (t2p[0, 1] + t2p[1, 1] + v[1]) + b2_ref[1]


def _dense3(t2p4, vp, d3, b2):
    return pl.pallas_call(
        _dense3_body,
        grid=(R // 8,),
        in_specs=[
            pl.BlockSpec((NCORES, 2, 8, 128), lambda i: (0, 0, i, 0)),
            pl.BlockSpec((2, 8, 128), lambda i: (0, i, 0)),
            pl.BlockSpec((8, 128), lambda i: (i, 0)),
            pl.BlockSpec(memory_space=pltpu.SMEM),
        ],
        out_specs=[
            pl.BlockSpec((8, 128), lambda i: (i, 0)),
            pl.BlockSpec((8, 128), lambda i: (i, 0)),
        ],
        out_shape=[
            jax.ShapeDtypeStruct((R, 128), jnp.float32),
            jax.ShapeDtypeStruct((R, 128), jnp.float32),
        ],
    )(t2p4, vp, d3, b2)


def kernel(x, edge_index, W1, b1, W2, b2):
    ei = edge_index.astype(jnp.int32)
    src1 = ei[0]
    dst1 = ei[1]
    xp = jnp.pad(x[:, 0], (0, NPAD - N))
    zero1 = jnp.zeros((NPAD,), jnp.float32)

    degp = _deg_pass(dst1, zero1)
    d3, u3 = _dense1(degp.reshape(NCORES, R, 128), xp.reshape(R, 128))

    tp = _agg1_pass(src1, dst1, u3.reshape(NPAD), zero1)
    vp = _dense2(tp.reshape(NCORES, R, 128), u3, d3, W1, b1, W2)

    t2p = _agg2_pass(src1, dst1, vp.reshape(2, NPAD), zero1)

    o0, o1 = _dense3(t2p.reshape(NCORES, 2, R, 128), vp, d3, b2)
    out = jnp.stack([o0.reshape(NPAD), o1.reshape(NPAD)], axis=-1)
    return out[:N]


# speedup vs baseline: 306.6856x; 1.2739x over previous
import functools

import jax
import jax.numpy as jnp
from jax import lax
from jax.experimental import pallas as pl
from jax.experimental.pallas import tpu as pltpu
from jax.experimental.pallas import tpu_sc as plsc

N = 100000
E = 6400000
NTILES = 16
NCORES = 2
NPAD = 100352
SLICE = NPAD // NTILES
R = NPAD // 128
CH = 2048
NCH = E // CH
CORE0 = (NCH + 1) // 2


def _chunk_range(cid, sid):
    per = jnp.where(cid == 0, CORE0, NCH - CORE0)
    base = cid * CORE0
    b = per // NTILES
    ex = per - b * NTILES
    start = base + sid * b + jnp.minimum(sid, ex)
    n = jnp.where(sid < ex, b + 1, b)
    return start, n


_SC_MESH = plsc.VectorSubcoreMesh(core_axis_name="c", subcore_axis_name="s")
_SC_PARAMS = pltpu.CompilerParams(use_tc_tiling_on_sc=False)


@functools.partial(
    pl.kernel,
    out_type=jax.ShapeDtypeStruct((NCORES, NPAD), jnp.float32),
    mesh=_SC_MESH,
    compiler_params=_SC_PARAMS,
    scratch_types=[
        pltpu.VMEM((CH,), jnp.int32),
        pltpu.VMEM((CH,), jnp.int32),
        pltpu.VMEM((CH,), jnp.int32),
        pltpu.VMEM((CH,), jnp.float32),
        pltpu.VMEM_SHARED((NPAD,), jnp.float32),
        pltpu.SemaphoreType.DMA,
        pltpu.SemaphoreType.DMA,
        pltpu.SemaphoreType.DMA,
        pltpu.SemaphoreType.DMA,
        pltpu.SemaphoreType.DMA,
        pltpu.SemaphoreType.DMA,
    ],
)
def _deg_pass(ei_h, zero_h, out_h, di0, di1, di2, ones_v, acc_s,
              semi0, semi1, semi2, sems0, sems1, sems2):
    didx = [di0, di1, di2]
    semi = [semi0, semi1, semi2]
    sems = [sems0, sems1, sems2]
    cid = lax.axis_index("c")
    sid = lax.axis_index("s")
    for i in range(CH // 16):
        ones_v[pl.ds(i * 16, 16)] = jnp.ones((16,), jnp.float32)
    sl = pl.ds(sid * SLICE, SLICE)
    pltpu.sync_copy(zero_h.at[sl], acc_s.at[sl])
    plsc.subcore_barrier()
    start, n = _chunk_range(cid, sid)

    def issue_idx(i, k):
        pltpu.async_copy(ei_h.at[1, pl.ds((start + i) * CH, CH)], didx[k], semi[k])

    issue_idx(0, 0)

    def body(g, carry):
        for k in range(3):
            i = g * 3 + k
            k2 = (k + 1) % 3

            @pl.when(i < n)
            def _(i=i, k=k, k2=k2):
                @pl.when(i >= 2)
                def _():
                    pltpu.make_async_copy(ones_v, acc_s.at[didx[k2]], sems[k2]).wait()

                @pl.when(i + 1 < n)
                def _():
                    issue_idx(i + 1, k2)

                pltpu.make_async_copy(ei_h.at[1, pl.ds(0, CH)], didx[k], semi[k]).wait()
                pltpu.async_copy(ones_v, acc_s.at[didx[k]], sems[k], add=True)

        return carry

    lax.fori_loop(0, (n + 2) // 3, body, 0)
    nm = n % 3
    for k in range(3):
        @pl.when(nm != k)
        def _(k=k):
            pltpu.make_async_copy(ones_v, acc_s.at[didx[k]], sems[k]).wait()
    plsc.subcore_barrier()
    pltpu.sync_copy(acc_s.at[sl], out_h.at[cid, sl])


@functools.partial(
    pl.kernel,
    out_type=jax.ShapeDtypeStruct((NCORES, NPAD), jnp.float32),
    mesh=_SC_MESH,
    compiler_params=_SC_PARAMS,
    scratch_types=[
        pltpu.VMEM((CH,), jnp.int32),
        pltpu.VMEM((CH,), jnp.int32),
        pltpu.VMEM((CH,), jnp.int32),
        pltpu.VMEM((CH,), jnp.int32),
        pltpu.VMEM((CH,), jnp.int32),
        pltpu.VMEM((CH,), jnp.int32),
        pltpu.VMEM((CH,), jnp.float32),
        pltpu.VMEM((CH,), jnp.float32),
        pltpu.VMEM((CH,), jnp.float32),
        pltpu.VMEM_SHARED((NPAD,), jnp.float32),
        pltpu.VMEM_SHARED((NPAD,), jnp.float32),
        pltpu.SemaphoreType.DMA,
        pltpu.SemaphoreType.DMA,
        pltpu.SemaphoreType.DMA,
        pltpu.SemaphoreType.DMA,
        pltpu.SemaphoreType.DMA,
        pltpu.SemaphoreType.DMA,
    ],
)
def _agg1_pass(ei_h, u_h, zero_h, out_h,
               si0, si1, si2, di0, di1, di2, va0, va1, va2, u_s, acc_s,
               semi0, semi1, semi2, sems0, sems1, sems2):
    sidx = [si0, si1, si2]
    didx = [di0, di1, di2]
    vals = [va0, va1, va2]
    semi = [semi0, semi1, semi2]
    sems = [sems0, sems1, sems2]
    cid = lax.axis_index("c")
    sid = lax.axis_index("s")
    sl = pl.ds(sid * SLICE, SLICE)
    pltpu.sync_copy(u_h.at[sl], u_s.at[sl])
    pltpu.sync_copy(zero_h.at[sl], acc_s.at[sl])
    plsc.subcore_barrier()
    start, n = _chunk_range(cid, sid)

    def issue_idx(i, k):
        off = (start + i) * CH
        pltpu.async_copy(ei_h.at[0, pl.ds(off, CH)], sidx[k], semi[k])
        pltpu.async_copy(ei_h.at[1, pl.ds(off, CH)], didx[k], semi[k])

    issue_idx(0, 0)

    def body(g, carry):
        for k in range(3):
            i = g * 3 + k
            k2 = (k + 1) % 3

            @pl.when(i < n)
            def _(i=i, k=k, k2=k2):
                @pl.when(i >= 2)
                def _():
                    pltpu.make_async_copy(vals[k2], acc_s.at[didx[k2]], sems[k2]).wait()

                @pl.when(i + 1 < n)
                def _():
                    issue_idx(i + 1, k2)

                pltpu.make_async_copy(ei_h.at[0, pl.ds(0, CH)], sidx[k], semi[k]).wait()
                pltpu.make_async_copy(ei_h.at[1, pl.ds(0, CH)], didx[k], semi[k]).wait()
                pltpu.sync_copy(u_s.at[sidx[k]], vals[k])
                pltpu.async_copy(vals[k], acc_s.at[didx[k]], sems[k], add=True)

        return carry

    lax.fori_loop(0, (n + 2) // 3, body, 0)
    nm = n % 3
    for k in range(3):
        @pl.when(nm != k)
        def _(k=k):
            pltpu.make_async_copy(vals[k], acc_s.at[didx[k]], sems[k]).wait()
    plsc.subcore_barrier()
    pltpu.sync_copy(acc_s.at[sl], out_h.at[cid, sl])


@functools.partial(
    pl.kernel,
    out_type=jax.ShapeDtypeStruct((NCORES, 2, NPAD), jnp.float32),
    mesh=_SC_MESH,
    compiler_params=_SC_PARAMS,
    scratch_types=[
        pltpu.VMEM((CH,), jnp.int32),
        pltpu.VMEM((CH,), jnp.int32),
        pltpu.VMEM((CH,), jnp.int32),
        pltpu.VMEM((CH,), jnp.int32),
        pltpu.VMEM((CH,), jnp.int32),
        pltpu.VMEM((CH,), jnp.int32),
        pltpu.VMEM((CH,), jnp.float32),
        pltpu.VMEM((CH,), jnp.float32),
        pltpu.VMEM((CH,), jnp.float32),
        pltpu.VMEM((CH,), jnp.float32),
        pltpu.VMEM((CH,), jnp.float32),
        pltpu.VMEM((CH,), jnp.float32),
        pltpu.VMEM_SHARED((NPAD,), jnp.float32),
        pltpu.VMEM_SHARED((NPAD,), jnp.float32),
        pltpu.VMEM_SHARED((NPAD,), jnp.float32),
        pltpu.VMEM_SHARED((NPAD,), jnp.float32),
        pltpu.SemaphoreType.DMA,
        pltpu.SemaphoreType.DMA,
        pltpu.SemaphoreType.DMA,
        pltpu.SemaphoreType.DMA,
        pltpu.SemaphoreType.DMA,
        pltpu.SemaphoreType.DMA,
        pltpu.SemaphoreType.DMA,
    ],
)
def _agg2_pass(ei_h, v_h, zero_h, out_h,
               si0, si1, si2, di0, di1, di2,
               va00, va01, va02, va10, va11, va12,
               v0_s, v1_s, acc0_s, acc1_s,
               semi0, semi1, semi2, sems0, sems1, sems2, semg):
    sidx = [si0, si1, si2]
    didx = [di0, di1, di2]
    vals0 = [va00, va01, va02]
    vals1 = [va10, va11, va12]
    semi = [semi0, semi1, semi2]
    sems = [sems0, sems1, sems2]
    cid = lax.axis_index("c")
    sid = lax.axis_index("s")
    sl = pl.ds(sid * SLICE, SLICE)
    pltpu.sync_copy(v_h.at[0, sl], v0_s.at[sl])
    pltpu.sync_copy(v_h.at[1, sl], v1_s.at[sl])
    pltpu.sync_copy(zero_h.at[sl], acc0_s.at[sl])
    pltpu.sync_copy(zero_h.at[sl], acc1_s.at[sl])
    plsc.subcore_barrier()
    start, n = _chunk_range(cid, sid)

    def issue_idx(i, k):
        off = (start + i) * CH
        pltpu.async_copy(ei_h.at[0, pl.ds(off, CH)], sidx[k], semi[k])
        pltpu.async_copy(ei_h.at[1, pl.ds(off, CH)], didx[k], semi[k])

    def wait_scatter(k):
        pltpu.make_async_copy(vals0[k], acc0_s.at[didx[k]], sems[k]).wait()
        pltpu.make_async_copy(vals1[k], acc1_s.at[didx[k]], sems[k]).wait()

    issue_idx(0, 0)

    def body(g, carry):
        for k in range(3):
            i = g * 3 + k
            k2 = (k + 1) % 3

            @pl.when(i < n)
            def _(i=i, k=k, k2=k2):
                @pl.when(i >= 2)
                def _():
                    wait_scatter(k2)

                @pl.when(i + 1 < n)
                def _():
                    issue_idx(i + 1, k2)

                pltpu.make_async_copy(ei_h.at[0, pl.ds(0, CH)], sidx[k], semi[k]).wait()
                pltpu.make_async_copy(ei_h.at[1, pl.ds(0, CH)], didx[k], semi[k]).wait()
                pltpu.async_copy(v0_s.at[sidx[k]], vals0[k], semg)
                pltpu.async_copy(v1_s.at[sidx[k]], vals1[k], semg)
                pltpu.make_async_copy(v0_s.at[sidx[k]], vals0[k], semg).wait()
                pltpu.make_async_copy(v1_s.at[sidx[k]], vals1[k], semg).wait()
                pltpu.async_copy(vals0[k], acc0_s.at[didx[k]], sems[k], add=True)
                pltpu.async_copy(vals1[k], acc1_s.at[didx[k]], sems[k], add=True)

        return carry

    lax.fori_loop(0, (n + 2) // 3, body, 0)
    nm = n % 3
    for k in range(3):
        @pl.when(nm != k)
        def _(k=k):
            wait_scatter(k)
    plsc.subcore_barrier()
    pltpu.sync_copy(acc0_s.at[sl], out_h.at[cid, 0, sl])
    pltpu.sync_copy(acc1_s.at[sl], out_h.at[cid, 1, sl])


def _dense1_body(degp_ref, x_ref, d_ref, u_ref):
    dp = degp_ref[...]
    deg = dp[0] + dp[1] + 1.0
    d = lax.rsqrt(deg)
    d_ref[...] = d
    u_ref[...] = d * x_ref[...]


def _dense1(degp3, xp3):
    return pl.pallas_call(
        _dense1_body,
        grid=(1,),
        in_specs=[
            pl.BlockSpec((NCORES, R, 128), lambda i: (0, 0, 0)),
            pl.BlockSpec((R, 128), lambda i: (0, 0)),
        ],
        out_specs=[
            pl.BlockSpec((R, 128), lambda i: (0, 0)),
            pl.BlockSpec((R, 128), lambda i: (0, 0)),
        ],
        out_shape=[
            jax.ShapeDtypeStruct((R, 128), jnp.float32),
            jax.ShapeDtypeStruct((R, 128), jnp.float32),
        ],
    )(degp3, xp3)


def _dense2_body(tp_ref, u_ref, d_ref, w1_ref, b1_ref, w2_ref, v_ref):
    tp = tp_ref[...]
    d = d_ref[...]
    s = d * (tp[0] + tp[1] + u_ref[...])
    z0 = jnp.zeros_like(s)
    z1 = jnp.zeros_like(s)
    for k in range(16):
        h = jnp.maximum(s * w1_ref[0, k] + b1_ref[k], 0.0)
        z0 = z0 + h * w2_ref[k, 0]
        z1 = z1 + h * w2_ref[k, 1]
    v_ref[0] = d * z0
    v_ref[1] = d * z1


def _dense2(tp3, u3, d3, w1, b1, w2):
    return pl.pallas_call(
        _dense2_body,
        grid=(1,),
        in_specs=[
            pl.BlockSpec((NCORES, R, 128), lambda i: (0, 0, 0)),
            pl.BlockSpec((R, 128), lambda i: (0, 0)),
            pl.BlockSpec((R, 128), lambda i: (0, 0)),
            pl.BlockSpec(memory_space=pltpu.SMEM),
            pl.BlockSpec(memory_space=pltpu.SMEM),
            pl.BlockSpec(memory_space=pltpu.SMEM),
        ],
        out_specs=pl.BlockSpec((2, R, 128), lambda i: (0, 0, 0)),
        out_shape=jax.ShapeDtypeStruct((2, R, 128), jnp.float32),
    )(tp3, u3, d3, w1, b1, w2)


def _dense3_body(t2p_ref, v_ref, d_ref, b2_ref, o0_ref, o1_ref):
    t2p = t2p_ref[...]
    v = v_ref[...]
    d = d_ref[...]
    o0_ref[...] = d * (t2p[0, 0] + t2p[1, 0] + v[0]) + b2_ref[0]
    o1_ref[...] = d * (t2p[0, 1] + t2p[1, 1] + v[1]) + b2_ref[1]


def _dense3(t2p4, vp, d3, b2):
    return pl.pallas_call(
        _dense3_body,
        grid=(1,),
        in_specs=[
            pl.BlockSpec((NCORES, 2, R, 128), lambda i: (0, 0, 0, 0)),
            pl.BlockSpec((2, R, 128), lambda i: (0, 0, 0)),
            pl.BlockSpec((R, 128), lambda i: (0, 0)),
            pl.BlockSpec(memory_space=pltpu.SMEM),
        ],
        out_specs=[
            pl.BlockSpec((R, 128), lambda i: (0, 0)),
            pl.BlockSpec((R, 128), lambda i: (0, 0)),
        ],
        out_shape=[
            jax.ShapeDtypeStruct((R, 128), jnp.float32),
            jax.ShapeDtypeStruct((R, 128), jnp.float32),
        ],
    )(t2p4, vp, d3, b2)


def kernel(x, edge_index, W1, b1, W2, b2):
    ei = edge_index.astype(jnp.int32)
    xp = jnp.pad(x[:, 0], (0, NPAD - N))
    zero1 = jnp.zeros((NPAD,), jnp.float32)

    degp = _deg_pass(ei, zero1)
    d3, u3 = _dense1(degp.reshape(NCORES, R, 128), xp.reshape(R, 128))

    tp = _agg1_pass(ei, u3.reshape(NPAD), zero1)
    vp = _dense2(tp.reshape(NCORES, R, 128), u3, d3, W1, b1, W2)

    t2p = _agg2_pass(ei, vp.reshape(2, NPAD), zero1)

    o0, o1 = _dense3(t2p.reshape(NCORES, 2, R, 128), vp, d3, b2)
    out = jnp.stack([o0.reshape(NPAD), o1.reshape(NPAD)], axis=-1)
    return out[:N]
